# baseline (speedup 1.0000x reference)
import functools

import jax
import jax.numpy as jnp
from jax import lax
from jax.experimental import pallas as pl
from jax.experimental.pallas import tpu as pltpu
from jax.experimental.pallas import tpu_sc as plsc

BATCH = 16384
NROWS = 1000000
EMBED_DIM = 64
FUSED_LANES = 2 * EMBED_DIM
NUM_CORES = 2
NUM_SUBCORES = 16
NUM_WORKERS = NUM_CORES * NUM_SUBCORES
B_PER_W = BATCH // NUM_WORKERS
BB = 256


@functools.cache
def _build():
    mesh = plsc.VectorSubcoreMesh(
        core_axis_name="c", subcore_axis_name="s",
        num_cores=NUM_CORES, num_subcores=NUM_SUBCORES)

    @functools.partial(
        pl.kernel,
        mesh=mesh,
        out_type=(
            jax.ShapeDtypeStruct((BATCH, FUSED_LANES), jnp.float32),
            jax.ShapeDtypeStruct((BATCH, FUSED_LANES), jnp.float32),
        ),
        scratch_types=[
            pltpu.VMEM((2 * B_PER_W,), jnp.int32),
            pltpu.VMEM((BB, FUSED_LANES), jnp.float32),
            pltpu.VMEM((BB, FUSED_LANES), jnp.float32),
            pltpu.SemaphoreType.DMA,
            pltpu.SemaphoreType.DMA,
            pltpu.SemaphoreType.DMA,
        ],
    )
    def gather2(qu_hbm, qi_hbm, fused_hbm, out_u, out_i,
                idx_v, rows0, rows1, gsem_a, gsem_b, wsem):
        wid = lax.axis_index("s") * NUM_CORES + lax.axis_index("c")
        base = wid * B_PER_W
        pltpu.sync_copy(qu_hbm.at[pl.ds(base, B_PER_W)],
                        idx_v.at[pl.ds(0, B_PER_W)])
        pltpu.sync_copy(qi_hbm.at[pl.ds(base, B_PER_W)],
                        idx_v.at[pl.ds(B_PER_W, B_PER_W)])

        rows = (rows0, rows1)
        gsems = (gsem_a, gsem_b)
        plan = [(out_u, 0), (out_u, BB), (out_i, B_PER_W), (out_i, B_PER_W + BB)]

        def fire(j):
            _, ioff = plan[j]
            pltpu.async_copy(
                fused_hbm.at[idx_v.at[pl.ds(ioff, BB)]], rows[j % 2],
                gsems[j % 2])

        fire(0)
        fire(1)
        for j in range(4):
            out, ioff = plan[j]
            pltpu.make_async_copy(
                fused_hbm.at[pl.ds(0, BB)], rows[j % 2], gsems[j % 2]).wait()
            pltpu.async_copy(rows[j % 2],
                             out.at[pl.ds(base + (ioff % B_PER_W), BB)], wsem)
            if j + 2 < 4:
                pltpu.make_async_copy(
                    out_u.at[pl.ds(0, BB)], rows[j % 2], wsem).wait()
                fire(j + 2)
        pltpu.make_async_copy(
            out_u.at[pl.ds(0, BB)], rows[0], wsem).wait()
        pltpu.make_async_copy(
            out_u.at[pl.ds(0, BB)], rows[1], wsem).wait()

    return gather2


def kernel(query_users, query_items, user_table, item_table):
    if query_users.ndim > 1:
        query_users = jnp.squeeze(query_users, axis=0)
    if query_items.ndim > 1:
        query_items = jnp.squeeze(query_items, axis=0)
    fused = jnp.concatenate([user_table, item_table], axis=1)
    u128, i128 = _build()(query_users.astype(jnp.int32),
                          query_items.astype(jnp.int32), fused)
    return (u128[:, :EMBED_DIM], i128[:, EMBED_DIM:])


# speedup vs baseline: 1.3017x; 1.3017x over previous
import functools

import jax
import jax.numpy as jnp
from jax import lax
from jax.experimental import pallas as pl
from jax.experimental.pallas import tpu as pltpu
from jax.experimental.pallas import tpu_sc as plsc

BATCH = 16384
EMBED_DIM = 64
NUM_CORES = 2
NUM_SUBCORES = 16
NUM_WORKERS = NUM_CORES * NUM_SUBCORES
B_PER_W = BATCH // NUM_WORKERS
WIN = 128
N_WIN = B_PER_W // WIN
LANES = 16


@functools.cache
def _build():
    mesh = plsc.VectorSubcoreMesh(
        core_axis_name="c", subcore_axis_name="s",
        num_cores=NUM_CORES, num_subcores=NUM_SUBCORES)

    @functools.partial(
        pl.kernel,
        mesh=mesh,
        out_type=(
            jax.ShapeDtypeStruct((BATCH, EMBED_DIM), jnp.float32),
            jax.ShapeDtypeStruct((BATCH, EMBED_DIM), jnp.float32),
        ),
        scratch_types=[
            pltpu.VMEM((2 * B_PER_W,), jnp.int32),
            pltpu.VMEM((B_PER_W, EMBED_DIM), jnp.float32),
            pltpu.SemaphoreType.DMA,
            pltpu.SemaphoreType.DMA,
            pltpu.SemaphoreType.DMA,
        ],
    )
    def gather2(qu_hbm, qi_hbm, ut_hbm, it_hbm, out_u, out_i,
                idx_v, rows_v, gsem_a, gsem_b, wsem):
        wid = lax.axis_index("s") * NUM_CORES + lax.axis_index("c")
        base = wid * B_PER_W
        pltpu.sync_copy(qu_hbm.at[pl.ds(base, B_PER_W)],
                        idx_v.at[pl.ds(0, B_PER_W)])
        pltpu.sync_copy(qi_hbm.at[pl.ds(base, B_PER_W)],
                        idx_v.at[pl.ds(B_PER_W, B_PER_W)])

        def fire_gathers(tbl, ioff, w, sem):
            def group(g, _):
                off = w * WIN + g * LANES
                v = idx_v[pl.ds(ioff + off, LANES)]
                for lane in range(LANES):
                    pltpu.async_copy(
                        tbl.at[pl.ds(v[lane], 1)],
                        rows_v.at[pl.ds(off + lane, 1)],
                        sem)
                return ()
            lax.fori_loop(0, WIN // LANES, group, ())

        def drain_g(sem):
            pltpu.make_async_copy(
                ut_hbm.at[pl.ds(0, WIN)],
                rows_v.at[pl.ds(0, WIN)], sem).wait()

        def fire_write(out, w):
            pltpu.async_copy(rows_v.at[pl.ds(w * WIN, WIN)],
                             out.at[pl.ds(base + w * WIN, WIN)], wsem)

        def drain_w():
            pltpu.make_async_copy(
                out_u.at[pl.ds(0, WIN)], rows_v.at[pl.ds(0, WIN)], wsem).wait()

        gsems = (gsem_a, gsem_b)
        for t, (tbl, out) in enumerate(((ut_hbm, out_u), (it_hbm, out_i))):
            ioff = t * B_PER_W
            for w in range(N_WIN):
                fire_gathers(tbl, ioff, w, gsems[w % 2])
                if w >= 1:
                    drain_g(gsems[(w - 1) % 2])
                    fire_write(out, w - 1)
            drain_g(gsems[(N_WIN - 1) % 2])
            fire_write(out, N_WIN - 1)
            for _ in range(N_WIN):
                drain_w()

    return gather2


def kernel(query_users, query_items, user_table, item_table):
    if query_users.ndim > 1:
        query_users = jnp.squeeze(query_users, axis=0)
    if query_items.ndim > 1:
        query_items = jnp.squeeze(query_items, axis=0)
    return _build()(query_users.astype(jnp.int32),
                    query_items.astype(jnp.int32),
                    user_table, item_table)
